# Initial kernel scaffold; baseline (speedup 1.0000x reference)
import jax
import jax.numpy as jnp
from jax.experimental import pallas as pl


def kernel(x, Wg, bg, W1, b1, W2, b2):
    raise NotImplementedError("write your pallas kernel here")


# speedup vs baseline: 2.1727x; 2.1727x over previous
import functools

import jax
import jax.numpy as jnp
from jax.experimental import pallas as pl
from jax.experimental.pallas import tpu as pltpu

B, L, H = 2, 2048, 768
F = 3072
E = 8
TOP_K = 2

BT = 2048
BF = 512


def _moe_dense_body(x_ref, wg_ref, bg_ref, w1_ref, b1_ref, w2_ref, b2_ref,
                    out_ref):
    e = pl.program_id(1)
    f = pl.program_id(2)
    xb = x_ref[...]

    logits = jnp.dot(xb, wg_ref[...],
                     preferred_element_type=jnp.float32) + bg_ref[...][None, :]
    cols = jax.lax.broadcasted_iota(jnp.int32, (BT, E), 1)
    m0 = jnp.max(logits, axis=-1, keepdims=True)
    i0 = jnp.min(jnp.where(logits == m0, cols, E), axis=-1, keepdims=True)
    masked = jnp.where(cols == i0, -jnp.inf, logits)
    m1 = jnp.max(masked, axis=-1, keepdims=True)
    i1 = jnp.min(jnp.where(masked == m1, cols, E), axis=-1, keepdims=True)
    t = jnp.exp(m1 - m0)
    s0 = 1.0 / (1.0 + t)
    s1 = t / (1.0 + t)
    we = jnp.where(i0 == e, s0, jnp.where(i1 == e, s1, 0.0))

    h = jnp.dot(xb, w1_ref[0], preferred_element_type=jnp.float32)
    h = h + b1_ref[0]
    h = 0.5 * h * (1.0 + jax.lax.erf(h * 0.7071067811865476))
    y = jnp.dot(h, w2_ref[0], preferred_element_type=jnp.float32) * we

    @pl.when(jnp.logical_and(e == 0, f == 0))
    def _init():
        out_ref[...] = jnp.zeros_like(out_ref)

    @pl.when(f == 0)
    def _bias():
        out_ref[...] = out_ref[...] + we * b2_ref[0]

    out_ref[...] = out_ref[...] + y


@functools.partial(jax.jit, static_argnames=("interpret",))
def kernel(x, Wg, bg, W1, b1, W2, b2, interpret=False):
    T = B * L
    x2 = x.reshape(T, H)
    out = pl.pallas_call(
        _moe_dense_body,
        grid=(T // BT, E, F // BF),
        in_specs=[
            pl.BlockSpec((BT, H), lambda i, e, f: (i, 0)),
            pl.BlockSpec((H, E), lambda i, e, f: (0, 0)),
            pl.BlockSpec((E,), lambda i, e, f: (0,)),
            pl.BlockSpec((1, H, BF), lambda i, e, f: (e, 0, f)),
            pl.BlockSpec((1, 1, BF), lambda i, e, f: (e, 0, f)),
            pl.BlockSpec((1, BF, H), lambda i, e, f: (e, f, 0)),
            pl.BlockSpec((1, 1, H), lambda i, e, f: (e, 0, 0)),
        ],
        out_specs=pl.BlockSpec((BT, H), lambda i, e, f: (i, 0)),
        out_shape=jax.ShapeDtypeStruct((T, H), jnp.float32),
        compiler_params=pltpu.CompilerParams(
            dimension_semantics=("arbitrary", "arbitrary", "arbitrary"),
        ),
        interpret=interpret,
    )(x2, Wg, bg, W1, b1.reshape(E, 1, F), W2, b2.reshape(E, 1, H))
    return out.reshape(B, L, H)


# speedup vs baseline: 5.2629x; 2.4223x over previous
import functools

import jax
import jax.numpy as jnp
from jax import lax
from jax.experimental import pallas as pl
from jax.experimental.pallas import tpu as pltpu
from jax.experimental.pallas import tpu_sc as plsc

B, L, H = 2, 2048, 768
F = 3072
E = 8
TOP_K = 2

T = B * L
A = TOP_K * T
BTF = 256
NBLK = A // BTF + E
S = NBLK * BTF

NC, NS = 2, 16
NW = NC * NS
APW = A // NW
SUB = 64
CH = 512


def _gate_body(x_ref, wg_ref, bg_ref, slot_ref, scores_ref, eob_ref,
               m_ref, rank_ref):
    xb = x_ref[...]
    logits = jnp.dot(xb, wg_ref[...],
                     preferred_element_type=jnp.float32) + bg_ref[...][None, :]
    cols = lax.broadcasted_iota(jnp.int32, (T, E), 1)
    m0 = jnp.max(logits, axis=-1, keepdims=True)
    i0 = jnp.min(jnp.where(logits == m0, cols, E), axis=-1, keepdims=True)
    masked = jnp.where(cols == i0, -jnp.inf, logits)
    m1 = jnp.max(masked, axis=-1, keepdims=True)
    i1 = jnp.min(jnp.where(masked == m1, cols, E), axis=-1, keepdims=True)
    t = jnp.exp(m1 - m0)
    s0 = 1.0 / (1.0 + t)
    s1 = t / (1.0 + t)
    scores_ref[...] = jnp.concatenate([s0, s1], axis=1)

    m_ref[pl.ds(0, T), :] = (cols == i0).astype(jnp.float32)
    m_ref[pl.ds(T, T), :] = (cols == i1).astype(jnp.float32)

    r = lax.broadcasted_iota(jnp.int32, (CH, CH), 0)
    c = lax.broadcasted_iota(jnp.int32, (CH, CH), 1)
    lstrict = (r > c).astype(jnp.float32)

    def body(ci, carry):
        blk = m_ref[pl.ds(ci * CH, CH), :]
        rank_ref[pl.ds(ci * CH, CH), :] = (
            jnp.dot(lstrict, blk, preferred_element_type=jnp.float32) + carry)
        return carry + jnp.sum(blk, axis=0, keepdims=True)

    counts = lax.fori_loop(0, A // CH, body, jnp.zeros((1, E), jnp.float32))

    nblk = jnp.ceil(counts / BTF)
    ri = lax.broadcasted_iota(jnp.int32, (E, E), 0)
    cj = lax.broadcasted_iota(jnp.int32, (E, E), 1)
    uincl = (ri <= cj).astype(jnp.float32)
    cb_incl = jnp.dot(nblk, uincl, preferred_element_type=jnp.float32)
    base_rows = (cb_incl - nblk) * BTF

    m = m_ref[...]
    rank_at = jnp.sum(rank_ref[...] * m, axis=1)
    base_at = jnp.sum(m * base_rows, axis=1)
    slot_ref[...] = (rank_at + base_at).astype(jnp.int32)

    bi = lax.broadcasted_iota(jnp.int32, (NBLK, E), 0)
    filled = (bi >= cb_incl.astype(jnp.int32)).astype(jnp.int32)
    eob_ref[...] = jnp.minimum(jnp.sum(filled, axis=1), E - 1)


def _gate(x2, Wg, bg, interpret=False):
    return pl.pallas_call(
        _gate_body,
        grid=(1,),
        in_specs=[
            pl.BlockSpec((T, H), lambda i: (0, 0)),
            pl.BlockSpec((H, E), lambda i: (0, 0)),
            pl.BlockSpec((E,), lambda i: (0,)),
        ],
        out_specs=[
            pl.BlockSpec((A,), lambda i: (0,)),
            pl.BlockSpec((T, TOP_K), lambda i: (0, 0)),
            pl.BlockSpec((NBLK,), lambda i: (0,)),
        ],
        out_shape=[
            jax.ShapeDtypeStruct((A,), jnp.int32),
            jax.ShapeDtypeStruct((T, TOP_K), jnp.float32),
            jax.ShapeDtypeStruct((NBLK,), jnp.int32),
        ],
        scratch_shapes=[
            pltpu.VMEM((A, E), jnp.float32),
            pltpu.VMEM((A, E), jnp.float32),
        ],
        interpret=interpret,
    )(x2, Wg, bg)


def _make_scatter():
    mesh = plsc.VectorSubcoreMesh(core_axis_name="c", subcore_axis_name="s",
                                  num_cores=NC, num_subcores=NS)

    @functools.partial(
        pl.kernel, mesh=mesh,
        out_type=jax.ShapeDtypeStruct((S, H), jnp.float32),
        scratch_types=[
            pltpu.VMEM((SUB,), jnp.int32),
            pltpu.VMEM((SUB, H), jnp.float32),
            pltpu.SemaphoreType.DMA,
        ],
    )
    def scatter_k(x_hbm, slot_hbm, xg_hbm, idx_v, rows_v, sem):
        wid = lax.axis_index("s") * NC + lax.axis_index("c")
        base = wid * APW
        for j in range(APW // SUB):
            a0 = base + j * SUB
            t0 = lax.rem(a0, T)
            pltpu.sync_copy(slot_hbm.at[pl.ds(a0, SUB)], idx_v)
            pltpu.sync_copy(x_hbm.at[pl.ds(t0, SUB), :], rows_v)
            pltpu.async_copy(rows_v, xg_hbm.at[idx_v], sem).wait()

    return scatter_k


def _ffn_body(eob_ref, xg_ref, w1_ref, b1_ref, w2_ref, b2_ref, yg_ref):
    h = jnp.dot(xg_ref[...], w1_ref[0], preferred_element_type=jnp.float32)
    h = h + b1_ref[0]
    h = 0.5 * h * (1.0 + lax.erf(h * 0.7071067811865476))
    yg_ref[...] = (jnp.dot(h, w2_ref[0], preferred_element_type=jnp.float32)
                   + b2_ref[0])


def _ffn(eob, xg, W1, b1r, W2, b2r, interpret=False):
    return pl.pallas_call(
        _ffn_body,
        grid_spec=pltpu.PrefetchScalarGridSpec(
            num_scalar_prefetch=1,
            grid=(NBLK,),
            in_specs=[
                pl.BlockSpec((BTF, H), lambda g, eob: (g, 0)),
                pl.BlockSpec((1, H, F), lambda g, eob: (eob[g], 0, 0)),
                pl.BlockSpec((1, 1, F), lambda g, eob: (eob[g], 0, 0)),
                pl.BlockSpec((1, F, H), lambda g, eob: (eob[g], 0, 0)),
                pl.BlockSpec((1, 1, H), lambda g, eob: (eob[g], 0, 0)),
            ],
            out_specs=pl.BlockSpec((BTF, H), lambda g, eob: (g, 0)),
        ),
        out_shape=jax.ShapeDtypeStruct((S, H), jnp.float32),
        compiler_params=pltpu.CompilerParams(
            dimension_semantics=("arbitrary",),
        ),
        interpret=interpret,
    )(eob, xg, W1, b1r, W2, b2r)


def _make_gather():
    mesh = plsc.VectorSubcoreMesh(core_axis_name="c", subcore_axis_name="s",
                                  num_cores=NC, num_subcores=NS)

    @functools.partial(
        pl.kernel, mesh=mesh,
        out_type=jax.ShapeDtypeStruct((A, H), jnp.float32),
        scratch_types=[
            pltpu.VMEM((SUB,), jnp.int32),
            pltpu.VMEM((SUB, H), jnp.float32),
            pltpu.SemaphoreType.DMA,
        ],
    )
    def gather_k(yg_hbm, slot_hbm, z_hbm, idx_v, rows_v, sem):
        wid = lax.axis_index("s") * NC + lax.axis_index("c")
        base = wid * APW
        for j in range(APW // SUB):
            a0 = base + j * SUB
            pltpu.sync_copy(slot_hbm.at[pl.ds(a0, SUB)], idx_v)
            pltpu.async_copy(yg_hbm.at[idx_v], rows_v, sem).wait()
            pltpu.sync_copy(rows_v, z_hbm.at[pl.ds(a0, SUB), :])

    return gather_k


_BC = 512


def _combine_body(z0_ref, z1_ref, sc_ref, out_ref):
    s = sc_ref[...]
    out_ref[...] = s[:, 0:1] * z0_ref[...] + s[:, 1:2] * z1_ref[...]


def _combine(z, scores, interpret=False):
    return pl.pallas_call(
        _combine_body,
        grid=(T // _BC,),
        in_specs=[
            pl.BlockSpec((_BC, H), lambda i: (i, 0)),
            pl.BlockSpec((_BC, H), lambda i: (i + T // _BC, 0)),
            pl.BlockSpec((_BC, TOP_K), lambda i: (i, 0)),
        ],
        out_specs=pl.BlockSpec((_BC, H), lambda i: (i, 0)),
        out_shape=jax.ShapeDtypeStruct((T, H), jnp.float32),
        interpret=interpret,
    )(z, z, scores)


_make_scatter = functools.cache(_make_scatter)
_make_gather = functools.cache(_make_gather)


@jax.jit
def kernel(x, Wg, bg, W1, b1, W2, b2):
    x2 = x.reshape(T, H)
    slot, scores, eob = _gate(x2, Wg, bg)
    xg = _make_scatter()(x2, slot)
    yg = _ffn(eob, xg, W1, b1.reshape(E, 1, F), W2, b2.reshape(E, 1, H))
    z = _make_gather()(yg, slot)
    out = _combine(z, scores)
    return out.reshape(B, L, H)


# speedup vs baseline: 5.3186x; 1.0106x over previous
import functools

import jax
import jax.numpy as jnp
from jax import lax
from jax.experimental import pallas as pl
from jax.experimental.pallas import tpu as pltpu
from jax.experimental.pallas import tpu_sc as plsc

B, L, H = 2, 2048, 768
F = 3072
E = 8
TOP_K = 2

T = B * L
A = TOP_K * T
BTF = 256
NBLK = A // BTF + E
S = NBLK * BTF

NC, NS = 2, 16
NW = NC * NS
APW = A // NW
SUB = 64
CH = 512


def _gate_body(x_ref, wg_ref, bg_ref, slot_ref, scores_ref, eob_ref,
               m_ref, rank_ref):
    xb = x_ref[...]
    logits = jnp.dot(xb, wg_ref[...],
                     preferred_element_type=jnp.float32) + bg_ref[...][None, :]
    cols = lax.broadcasted_iota(jnp.int32, (T, E), 1)
    m0 = jnp.max(logits, axis=-1, keepdims=True)
    i0 = jnp.min(jnp.where(logits == m0, cols, E), axis=-1, keepdims=True)
    masked = jnp.where(cols == i0, -jnp.inf, logits)
    m1 = jnp.max(masked, axis=-1, keepdims=True)
    i1 = jnp.min(jnp.where(masked == m1, cols, E), axis=-1, keepdims=True)
    t = jnp.exp(m1 - m0)
    s0 = 1.0 / (1.0 + t)
    s1 = t / (1.0 + t)
    scores_ref[...] = jnp.concatenate([s0, s1], axis=1)

    m_ref[pl.ds(0, T), :] = (cols == i0).astype(jnp.float32)
    m_ref[pl.ds(T, T), :] = (cols == i1).astype(jnp.float32)

    r = lax.broadcasted_iota(jnp.int32, (CH, CH), 0)
    c = lax.broadcasted_iota(jnp.int32, (CH, CH), 1)
    lstrict = (r > c).astype(jnp.float32)

    def body(ci, carry):
        blk = m_ref[pl.ds(ci * CH, CH), :]
        rank_ref[pl.ds(ci * CH, CH), :] = (
            jnp.dot(lstrict, blk, preferred_element_type=jnp.float32) + carry)
        return carry + jnp.sum(blk, axis=0, keepdims=True)

    counts = lax.fori_loop(0, A // CH, body, jnp.zeros((1, E), jnp.float32))

    nblk = jnp.ceil(counts / BTF)
    ri = lax.broadcasted_iota(jnp.int32, (E, E), 0)
    cj = lax.broadcasted_iota(jnp.int32, (E, E), 1)
    uincl = (ri <= cj).astype(jnp.float32)
    cb_incl = jnp.dot(nblk, uincl, preferred_element_type=jnp.float32)
    base_rows = (cb_incl - nblk) * BTF

    m = m_ref[...]
    rank_at = jnp.sum(rank_ref[...] * m, axis=1)
    base_at = jnp.sum(m * base_rows, axis=1)
    slot_ref[...] = (rank_at + base_at).astype(jnp.int32)

    bi = lax.broadcasted_iota(jnp.int32, (NBLK, E), 0)
    filled = (bi >= cb_incl.astype(jnp.int32)).astype(jnp.int32)
    eob_ref[pl.ds(0, NBLK)] = jnp.minimum(jnp.sum(filled, axis=1), E - 1)
    eob_ref[pl.ds(NBLK, 1)] = jnp.sum(nblk, axis=1).astype(jnp.int32)


def _gate(x2, Wg, bg, interpret=False):
    return pl.pallas_call(
        _gate_body,
        grid=(1,),
        in_specs=[
            pl.BlockSpec((T, H), lambda i: (0, 0)),
            pl.BlockSpec((H, E), lambda i: (0, 0)),
            pl.BlockSpec((E,), lambda i: (0,)),
        ],
        out_specs=[
            pl.BlockSpec((A,), lambda i: (0,)),
            pl.BlockSpec((T, TOP_K), lambda i: (0, 0)),
            pl.BlockSpec((NBLK + 1,), lambda i: (0,)),
        ],
        out_shape=[
            jax.ShapeDtypeStruct((A,), jnp.int32),
            jax.ShapeDtypeStruct((T, TOP_K), jnp.float32),
            jax.ShapeDtypeStruct((NBLK + 1,), jnp.int32),
        ],
        scratch_shapes=[
            pltpu.VMEM((A, E), jnp.float32),
            pltpu.VMEM((A, E), jnp.float32),
        ],
        interpret=interpret,
    )(x2, Wg, bg)


def _make_scatter():
    mesh = plsc.VectorSubcoreMesh(core_axis_name="c", subcore_axis_name="s",
                                  num_cores=NC, num_subcores=NS)

    @functools.partial(
        pl.kernel, mesh=mesh,
        out_type=jax.ShapeDtypeStruct((S, H), jnp.float32),
        scratch_types=[
            pltpu.VMEM((SUB,), jnp.int32),
            pltpu.VMEM((SUB, H), jnp.float32),
            pltpu.SemaphoreType.DMA,
        ],
    )
    def scatter_k(x_hbm, slot_hbm, xg_hbm, idx_v, rows_v, sem):
        wid = lax.axis_index("s") * NC + lax.axis_index("c")
        base = wid * APW
        for j in range(APW // SUB):
            a0 = base + j * SUB
            t0 = lax.rem(a0, T)
            pltpu.sync_copy(slot_hbm.at[pl.ds(a0, SUB)], idx_v)
            pltpu.sync_copy(x_hbm.at[pl.ds(t0, SUB), :], rows_v)
            pltpu.async_copy(rows_v, xg_hbm.at[idx_v], sem).wait()

    return scatter_k


def _ffn_body(eob_ref, xg_ref, w1_ref, b1_ref, w2_ref, b2_ref, yg_ref):
    g = pl.program_id(0)

    @pl.when(g < eob_ref[NBLK])
    def _active():
        xb = xg_ref[...].astype(jnp.bfloat16)
        h = jnp.dot(xb, w1_ref[0].astype(jnp.bfloat16),
                    preferred_element_type=jnp.float32)
        h = h + b1_ref[0]
        h = 0.5 * h * (1.0 + lax.erf(h * 0.7071067811865476))
        yg_ref[...] = (jnp.dot(h.astype(jnp.bfloat16),
                               w2_ref[0].astype(jnp.bfloat16),
                               preferred_element_type=jnp.float32)
                       + b2_ref[0])


def _ffn(eob, xg, W1, b1r, W2, b2r, interpret=False):
    return pl.pallas_call(
        _ffn_body,
        grid_spec=pltpu.PrefetchScalarGridSpec(
            num_scalar_prefetch=1,
            grid=(NBLK,),
            in_specs=[
                pl.BlockSpec((BTF, H), lambda g, eob: (g, 0)),
                pl.BlockSpec((1, H, F), lambda g, eob: (eob[g], 0, 0)),
                pl.BlockSpec((1, 1, F), lambda g, eob: (eob[g], 0, 0)),
                pl.BlockSpec((1, F, H), lambda g, eob: (eob[g], 0, 0)),
                pl.BlockSpec((1, 1, H), lambda g, eob: (eob[g], 0, 0)),
            ],
            out_specs=pl.BlockSpec((BTF, H), lambda g, eob: (g, 0)),
        ),
        out_shape=jax.ShapeDtypeStruct((S, H), jnp.float32),
        compiler_params=pltpu.CompilerParams(
            dimension_semantics=("arbitrary",),
        ),
        interpret=interpret,
    )(eob, xg, W1, b1r, W2, b2r)


def _make_gather():
    mesh = plsc.VectorSubcoreMesh(core_axis_name="c", subcore_axis_name="s",
                                  num_cores=NC, num_subcores=NS)

    @functools.partial(
        pl.kernel, mesh=mesh,
        out_type=jax.ShapeDtypeStruct((A, H), jnp.float32),
        scratch_types=[
            pltpu.VMEM((SUB,), jnp.int32),
            pltpu.VMEM((SUB, H), jnp.float32),
            pltpu.SemaphoreType.DMA,
        ],
    )
    def gather_k(yg_hbm, slot_hbm, z_hbm, idx_v, rows_v, sem):
        wid = lax.axis_index("s") * NC + lax.axis_index("c")
        base = wid * APW
        for j in range(APW // SUB):
            a0 = base + j * SUB
            pltpu.sync_copy(slot_hbm.at[pl.ds(a0, SUB)], idx_v)
            pltpu.async_copy(yg_hbm.at[idx_v], rows_v, sem).wait()
            pltpu.sync_copy(rows_v, z_hbm.at[pl.ds(a0, SUB), :])

    return gather_k


_BC = 512


def _combine_body(z0_ref, z1_ref, sc_ref, out_ref):
    s = sc_ref[...]
    out_ref[...] = s[:, 0:1] * z0_ref[...] + s[:, 1:2] * z1_ref[...]


def _combine(z, scores, interpret=False):
    return pl.pallas_call(
        _combine_body,
        grid=(T // _BC,),
        in_specs=[
            pl.BlockSpec((_BC, H), lambda i: (i, 0)),
            pl.BlockSpec((_BC, H), lambda i: (i + T // _BC, 0)),
            pl.BlockSpec((_BC, TOP_K), lambda i: (i, 0)),
        ],
        out_specs=pl.BlockSpec((_BC, H), lambda i: (i, 0)),
        out_shape=jax.ShapeDtypeStruct((T, H), jnp.float32),
        interpret=interpret,
    )(z, z, scores)


_make_scatter = functools.cache(_make_scatter)
_make_gather = functools.cache(_make_gather)


@jax.jit
def kernel(x, Wg, bg, W1, b1, W2, b2):
    x2 = x.reshape(T, H)
    slot, scores, eob = _gate(x2, Wg, bg)
    xg = _make_scatter()(x2, slot)
    yg = _ffn(eob, xg, W1, b1.reshape(E, 1, F), W2, b2.reshape(E, 1, H))
    z = _make_gather()(yg, slot)
    out = _combine(z, scores)
    return out.reshape(B, L, H)


# speedup vs baseline: 5.4997x; 1.0341x over previous
import functools

import jax
import jax.numpy as jnp
from jax import lax
from jax.experimental import pallas as pl
from jax.experimental.pallas import tpu as pltpu
from jax.experimental.pallas import tpu_sc as plsc

B, L, H = 2, 2048, 768
F = 3072
E = 8
TOP_K = 2

T = B * L
A = TOP_K * T
BTF = 256
NBLK = A // BTF + E
S = NBLK * BTF

NC, NS = 2, 16
NW = NC * NS
APW = A // NW
SUB = 64
CH = 512


def _gate_body(x_ref, wg_ref, bg_ref, slot_ref, scores_ref, eob_ref,
               m_ref, rank_ref):
    xb = x_ref[...]
    logits = jnp.dot(xb, wg_ref[...],
                     preferred_element_type=jnp.float32) + bg_ref[...][None, :]
    cols = lax.broadcasted_iota(jnp.int32, (T, E), 1)
    m0 = jnp.max(logits, axis=-1, keepdims=True)
    i0 = jnp.min(jnp.where(logits == m0, cols, E), axis=-1, keepdims=True)
    masked = jnp.where(cols == i0, -jnp.inf, logits)
    m1 = jnp.max(masked, axis=-1, keepdims=True)
    i1 = jnp.min(jnp.where(masked == m1, cols, E), axis=-1, keepdims=True)
    t = jnp.exp(m1 - m0)
    s0 = 1.0 / (1.0 + t)
    s1 = t / (1.0 + t)
    scores_ref[...] = jnp.concatenate([s0, s1], axis=1)

    m_ref[pl.ds(0, T), :] = (cols == i0).astype(jnp.float32)
    m_ref[pl.ds(T, T), :] = (cols == i1).astype(jnp.float32)

    r = lax.broadcasted_iota(jnp.int32, (CH, CH), 0)
    c = lax.broadcasted_iota(jnp.int32, (CH, CH), 1)
    lstrict = (r > c).astype(jnp.float32)

    def body(ci, carry):
        blk = m_ref[pl.ds(ci * CH, CH), :]
        rank_ref[pl.ds(ci * CH, CH), :] = (
            jnp.dot(lstrict, blk, preferred_element_type=jnp.float32) + carry)
        return carry + jnp.sum(blk, axis=0, keepdims=True)

    counts = lax.fori_loop(0, A // CH, body, jnp.zeros((1, E), jnp.float32))

    nblk = jnp.ceil(counts / BTF)
    ri = lax.broadcasted_iota(jnp.int32, (E, E), 0)
    cj = lax.broadcasted_iota(jnp.int32, (E, E), 1)
    uincl = (ri <= cj).astype(jnp.float32)
    cb_incl = jnp.dot(nblk, uincl, preferred_element_type=jnp.float32)
    base_rows = (cb_incl - nblk) * BTF

    m = m_ref[...]
    rank_at = jnp.sum(rank_ref[...] * m, axis=1)
    base_at = jnp.sum(m * base_rows, axis=1)
    slot_ref[...] = (rank_at + base_at).astype(jnp.int32)

    bi = lax.broadcasted_iota(jnp.int32, (NBLK, E), 0)
    filled = (bi >= cb_incl.astype(jnp.int32)).astype(jnp.int32)
    eob_ref[pl.ds(0, NBLK)] = jnp.minimum(jnp.sum(filled, axis=1), E - 1)
    eob_ref[pl.ds(NBLK, 1)] = jnp.sum(nblk, axis=1).astype(jnp.int32)


def _gate(x2, Wg, bg, interpret=False):
    return pl.pallas_call(
        _gate_body,
        grid=(1,),
        in_specs=[
            pl.BlockSpec((T, H), lambda i: (0, 0)),
            pl.BlockSpec((H, E), lambda i: (0, 0)),
            pl.BlockSpec((E,), lambda i: (0,)),
        ],
        out_specs=[
            pl.BlockSpec((A,), lambda i: (0,)),
            pl.BlockSpec((T, TOP_K), lambda i: (0, 0)),
            pl.BlockSpec((NBLK + 1,), lambda i: (0,)),
        ],
        out_shape=[
            jax.ShapeDtypeStruct((A,), jnp.int32),
            jax.ShapeDtypeStruct((T, TOP_K), jnp.float32),
            jax.ShapeDtypeStruct((NBLK + 1,), jnp.int32),
        ],
        scratch_shapes=[
            pltpu.VMEM((A, E), jnp.float32),
            pltpu.VMEM((A, E), jnp.float32),
        ],
        interpret=interpret,
    )(x2, Wg, bg)


def _make_scatter():
    mesh = plsc.VectorSubcoreMesh(core_axis_name="c", subcore_axis_name="s",
                                  num_cores=NC, num_subcores=NS)

    nch = APW // SUB

    @functools.partial(
        pl.kernel, mesh=mesh,
        out_type=jax.ShapeDtypeStruct((S, H), jnp.float32),
        scratch_types=[
            pltpu.VMEM((nch, SUB), jnp.int32),
            pltpu.VMEM((2, SUB, H), jnp.float32),
            pltpu.SemaphoreType.DMA,
            pltpu.SemaphoreType.DMA,
            pltpu.SemaphoreType.DMA,
            pltpu.SemaphoreType.DMA,
        ],
    )
    def scatter_k(x_hbm, slot2_hbm, xg_hbm, idx_v, rows_v, ls0, ls1, ss0, ss1):
        wid = lax.axis_index("s") * NC + lax.axis_index("c")
        base = wid * APW
        pltpu.sync_copy(slot2_hbm.at[pl.ds(wid * nch, nch), :], idx_v)
        lsem = [ls0, ls1]
        ssem = [ss0, ss1]

        def load(j):
            t0 = lax.rem(base + j * SUB, T)
            return pltpu.async_copy(x_hbm.at[pl.ds(t0, SUB), :],
                                    rows_v.at[j % 2], lsem[j % 2])

        def scat(j):
            return pltpu.async_copy(rows_v.at[j % 2], xg_hbm.at[idx_v.at[j]],
                                    ssem[j % 2])

        l0 = load(0)
        l1 = load(1)
        l0.wait()
        s0 = scat(0)
        l1.wait()
        s1 = scat(1)
        s0.wait()
        l2 = load(2)
        s1.wait()
        l3 = load(3)
        l2.wait()
        s2 = scat(2)
        l3.wait()
        s3 = scat(3)
        s2.wait()
        s3.wait()

    return scatter_k


def _ffn_body(eob_ref, xg_ref, w1_ref, b1_ref, w2_ref, b2_ref, yg_ref):
    g = pl.program_id(0)

    @pl.when(g < eob_ref[NBLK])
    def _active():
        h = jnp.dot(xg_ref[...], w1_ref[0],
                    preferred_element_type=jnp.float32)
        h = h + b1_ref[0]
        h = 0.5 * h * (1.0 + lax.erf(h * 0.7071067811865476))
        yg_ref[...] = (jnp.dot(h, w2_ref[0],
                               preferred_element_type=jnp.float32)
                       + b2_ref[0])


def _ffn(eob, xg, W1, b1r, W2, b2r, interpret=False):
    return pl.pallas_call(
        _ffn_body,
        grid_spec=pltpu.PrefetchScalarGridSpec(
            num_scalar_prefetch=1,
            grid=(NBLK,),
            in_specs=[
                pl.BlockSpec((BTF, H), lambda g, eob: (g, 0)),
                pl.BlockSpec((1, H, F), lambda g, eob: (eob[g], 0, 0)),
                pl.BlockSpec((1, 1, F), lambda g, eob: (eob[g], 0, 0)),
                pl.BlockSpec((1, F, H), lambda g, eob: (eob[g], 0, 0)),
                pl.BlockSpec((1, 1, H), lambda g, eob: (eob[g], 0, 0)),
            ],
            out_specs=pl.BlockSpec((BTF, H), lambda g, eob: (g, 0)),
        ),
        out_shape=jax.ShapeDtypeStruct((S, H), jnp.float32),
        compiler_params=pltpu.CompilerParams(
            dimension_semantics=("arbitrary",),
        ),
        interpret=interpret,
    )(eob, xg, W1, b1r, W2, b2r)


def _make_gather():
    mesh = plsc.VectorSubcoreMesh(core_axis_name="c", subcore_axis_name="s",
                                  num_cores=NC, num_subcores=NS)

    nch = APW // SUB

    @functools.partial(
        pl.kernel, mesh=mesh,
        out_type=jax.ShapeDtypeStruct((A, H), jnp.float32),
        scratch_types=[
            pltpu.VMEM((nch, SUB), jnp.int32),
            pltpu.VMEM((2, SUB, H), jnp.float32),
            pltpu.SemaphoreType.DMA,
            pltpu.SemaphoreType.DMA,
            pltpu.SemaphoreType.DMA,
            pltpu.SemaphoreType.DMA,
        ],
    )
    def gather_k(yg_hbm, slot2_hbm, z_hbm, idx_v, rows_v, gs0, gs1, ws0, ws1):
        wid = lax.axis_index("s") * NC + lax.axis_index("c")
        base = wid * APW
        pltpu.sync_copy(slot2_hbm.at[pl.ds(wid * nch, nch), :], idx_v)
        gsem = [gs0, gs1]
        wsem = [ws0, ws1]

        def gath(j):
            return pltpu.async_copy(yg_hbm.at[idx_v.at[j]], rows_v.at[j % 2],
                                    gsem[j % 2])

        def store(j):
            a0 = base + j * SUB
            return pltpu.async_copy(rows_v.at[j % 2],
                                    z_hbm.at[pl.ds(a0, SUB), :], wsem[j % 2])

        g0 = gath(0)
        g1 = gath(1)
        g0.wait()
        w0 = store(0)
        g1.wait()
        w1 = store(1)
        w0.wait()
        g2 = gath(2)
        w1.wait()
        g3 = gath(3)
        g2.wait()
        w2 = store(2)
        g3.wait()
        w3 = store(3)
        w2.wait()
        w3.wait()

    return gather_k


_BC = 512


def _combine_body(z0_ref, z1_ref, sc_ref, out_ref):
    s = sc_ref[...]
    out_ref[...] = s[:, 0:1] * z0_ref[...] + s[:, 1:2] * z1_ref[...]


def _combine(z, scores, interpret=False):
    return pl.pallas_call(
        _combine_body,
        grid=(T // _BC,),
        in_specs=[
            pl.BlockSpec((_BC, H), lambda i: (i, 0)),
            pl.BlockSpec((_BC, H), lambda i: (i + T // _BC, 0)),
            pl.BlockSpec((_BC, TOP_K), lambda i: (i, 0)),
        ],
        out_specs=pl.BlockSpec((_BC, H), lambda i: (i, 0)),
        out_shape=jax.ShapeDtypeStruct((T, H), jnp.float32),
        interpret=interpret,
    )(z, z, scores)


_make_scatter = functools.cache(_make_scatter)
_make_gather = functools.cache(_make_gather)


@jax.jit
def kernel(x, Wg, bg, W1, b1, W2, b2):
    x2 = x.reshape(T, H)
    slot, scores, eob = _gate(x2, Wg, bg)
    slot2 = slot.reshape(A // SUB, SUB)
    xg = _make_scatter()(x2, slot2)
    yg = _ffn(eob, xg, W1, b1.reshape(E, 1, F), W2, b2.reshape(E, 1, H))
    z = _make_gather()(yg, slot2)
    out = _combine(z, scores)
    return out.reshape(B, L, H)


# speedup vs baseline: 5.7510x; 1.0457x over previous
import functools

import jax
import jax.numpy as jnp
from jax import lax
from jax.experimental import pallas as pl
from jax.experimental.pallas import tpu as pltpu
from jax.experimental.pallas import tpu_sc as plsc

B, L, H = 2, 2048, 768
F = 3072
E = 8
TOP_K = 2

T = B * L
A = TOP_K * T
BTF = 512
NBLK = A // BTF + E
S = NBLK * BTF

NC, NS = 2, 16
NW = NC * NS
APW = A // NW
SUB = 64
CH = 512


def _gate_body(x_ref, wg_ref, bg_ref, slot_ref, scores_ref, eob_ref,
               m_ref, rank_ref):
    xb = x_ref[...]
    logits = jnp.dot(xb, wg_ref[...],
                     preferred_element_type=jnp.float32) + bg_ref[...][None, :]
    cols = lax.broadcasted_iota(jnp.int32, (T, E), 1)
    m0 = jnp.max(logits, axis=-1, keepdims=True)
    i0 = jnp.min(jnp.where(logits == m0, cols, E), axis=-1, keepdims=True)
    masked = jnp.where(cols == i0, -jnp.inf, logits)
    m1 = jnp.max(masked, axis=-1, keepdims=True)
    i1 = jnp.min(jnp.where(masked == m1, cols, E), axis=-1, keepdims=True)
    t = jnp.exp(m1 - m0)
    s0 = 1.0 / (1.0 + t)
    s1 = t / (1.0 + t)
    scores_ref[...] = jnp.concatenate([s0, s1], axis=1)

    m_ref[pl.ds(0, T), :] = (cols == i0).astype(jnp.float32)
    m_ref[pl.ds(T, T), :] = (cols == i1).astype(jnp.float32)

    r = lax.broadcasted_iota(jnp.int32, (CH, CH), 0)
    c = lax.broadcasted_iota(jnp.int32, (CH, CH), 1)
    lstrict = (r > c).astype(jnp.float32)

    def body(ci, carry):
        blk = m_ref[pl.ds(ci * CH, CH), :]
        rank_ref[pl.ds(ci * CH, CH), :] = (
            jnp.dot(lstrict, blk, preferred_element_type=jnp.float32) + carry)
        return carry + jnp.sum(blk, axis=0, keepdims=True)

    counts = lax.fori_loop(0, A // CH, body, jnp.zeros((1, E), jnp.float32))

    nblk = jnp.ceil(counts / BTF)
    ri = lax.broadcasted_iota(jnp.int32, (E, E), 0)
    cj = lax.broadcasted_iota(jnp.int32, (E, E), 1)
    uincl = (ri <= cj).astype(jnp.float32)
    cb_incl = jnp.dot(nblk, uincl, preferred_element_type=jnp.float32)
    base_rows = (cb_incl - nblk) * BTF

    m = m_ref[...]
    rank_at = jnp.sum(rank_ref[...] * m, axis=1)
    base_at = jnp.sum(m * base_rows, axis=1)
    slot_ref[...] = (rank_at + base_at).astype(jnp.int32)

    bi = lax.broadcasted_iota(jnp.int32, (NBLK, E), 0)
    filled = (bi >= cb_incl.astype(jnp.int32)).astype(jnp.int32)
    eob_ref[pl.ds(0, NBLK)] = jnp.minimum(jnp.sum(filled, axis=1), E - 1)
    eob_ref[pl.ds(NBLK, 1)] = jnp.sum(nblk, axis=1).astype(jnp.int32)


def _gate(x2, Wg, bg, interpret=False):
    return pl.pallas_call(
        _gate_body,
        grid=(1,),
        in_specs=[
            pl.BlockSpec((T, H), lambda i: (0, 0)),
            pl.BlockSpec((H, E), lambda i: (0, 0)),
            pl.BlockSpec((E,), lambda i: (0,)),
        ],
        out_specs=[
            pl.BlockSpec((A,), lambda i: (0,)),
            pl.BlockSpec((T, TOP_K), lambda i: (0, 0)),
            pl.BlockSpec((NBLK + 1,), lambda i: (0,)),
        ],
        out_shape=[
            jax.ShapeDtypeStruct((A,), jnp.int32),
            jax.ShapeDtypeStruct((T, TOP_K), jnp.float32),
            jax.ShapeDtypeStruct((NBLK + 1,), jnp.int32),
        ],
        scratch_shapes=[
            pltpu.VMEM((A, E), jnp.float32),
            pltpu.VMEM((A, E), jnp.float32),
        ],
        interpret=interpret,
    )(x2, Wg, bg)


def _make_scatter():
    mesh = plsc.VectorSubcoreMesh(core_axis_name="c", subcore_axis_name="s",
                                  num_cores=NC, num_subcores=NS)

    nch = APW // SUB

    @functools.partial(
        pl.kernel, mesh=mesh,
        out_type=jax.ShapeDtypeStruct((S, H), jnp.float32),
        scratch_types=[
            pltpu.VMEM((nch, SUB), jnp.int32),
            pltpu.VMEM((2, SUB, H), jnp.float32),
            pltpu.SemaphoreType.DMA,
            pltpu.SemaphoreType.DMA,
            pltpu.SemaphoreType.DMA,
            pltpu.SemaphoreType.DMA,
        ],
    )
    def scatter_k(x_hbm, slot2_hbm, xg_hbm, idx_v, rows_v, ls0, ls1, ss0, ss1):
        wid = lax.axis_index("s") * NC + lax.axis_index("c")
        base = wid * APW
        pltpu.sync_copy(slot2_hbm.at[pl.ds(wid * nch, nch), :], idx_v)
        lsem = [ls0, ls1]
        ssem = [ss0, ss1]

        def load(j):
            t0 = lax.rem(base + j * SUB, T)
            return pltpu.async_copy(x_hbm.at[pl.ds(t0, SUB), :],
                                    rows_v.at[j % 2], lsem[j % 2])

        def scat(j):
            return pltpu.async_copy(rows_v.at[j % 2], xg_hbm.at[idx_v.at[j]],
                                    ssem[j % 2])

        l0 = load(0)
        l1 = load(1)
        l0.wait()
        s0 = scat(0)
        l1.wait()
        s1 = scat(1)
        s0.wait()
        l2 = load(2)
        s1.wait()
        l3 = load(3)
        l2.wait()
        s2 = scat(2)
        l3.wait()
        s3 = scat(3)
        s2.wait()
        s3.wait()

    return scatter_k


def _ffn_body(eob_ref, xg_ref, w1_ref, b1_ref, w2_ref, b2_ref, yg_ref):
    g = pl.program_id(0)

    @pl.when(g < eob_ref[NBLK])
    def _active():
        h = jnp.dot(xg_ref[...], w1_ref[0],
                    preferred_element_type=jnp.float32)
        h = h + b1_ref[0]
        h = 0.5 * h * (1.0 + lax.erf(h * 0.7071067811865476))
        yg_ref[...] = (jnp.dot(h, w2_ref[0],
                               preferred_element_type=jnp.float32)
                       + b2_ref[0])


def _ffn(eob, xg, W1, b1r, W2, b2r, interpret=False):
    return pl.pallas_call(
        _ffn_body,
        grid_spec=pltpu.PrefetchScalarGridSpec(
            num_scalar_prefetch=1,
            grid=(NBLK,),
            in_specs=[
                pl.BlockSpec((BTF, H), lambda g, eob: (g, 0)),
                pl.BlockSpec((1, H, F), lambda g, eob: (eob[g], 0, 0)),
                pl.BlockSpec((1, 1, F), lambda g, eob: (eob[g], 0, 0)),
                pl.BlockSpec((1, F, H), lambda g, eob: (eob[g], 0, 0)),
                pl.BlockSpec((1, 1, H), lambda g, eob: (eob[g], 0, 0)),
            ],
            out_specs=pl.BlockSpec((BTF, H), lambda g, eob: (g, 0)),
        ),
        out_shape=jax.ShapeDtypeStruct((S, H), jnp.float32),
        compiler_params=pltpu.CompilerParams(
            dimension_semantics=("arbitrary",),
        ),
        interpret=interpret,
    )(eob, xg, W1, b1r, W2, b2r)


def _make_gather():
    mesh = plsc.VectorSubcoreMesh(core_axis_name="c", subcore_axis_name="s",
                                  num_cores=NC, num_subcores=NS)

    nch = APW // SUB

    @functools.partial(
        pl.kernel, mesh=mesh,
        out_type=jax.ShapeDtypeStruct((A, H), jnp.float32),
        scratch_types=[
            pltpu.VMEM((nch, SUB), jnp.int32),
            pltpu.VMEM((2, SUB, H), jnp.float32),
            pltpu.SemaphoreType.DMA,
            pltpu.SemaphoreType.DMA,
            pltpu.SemaphoreType.DMA,
            pltpu.SemaphoreType.DMA,
        ],
    )
    def gather_k(yg_hbm, slot2_hbm, z_hbm, idx_v, rows_v, gs0, gs1, ws0, ws1):
        wid = lax.axis_index("s") * NC + lax.axis_index("c")
        base = wid * APW
        pltpu.sync_copy(slot2_hbm.at[pl.ds(wid * nch, nch), :], idx_v)
        gsem = [gs0, gs1]
        wsem = [ws0, ws1]

        def gath(j):
            return pltpu.async_copy(yg_hbm.at[idx_v.at[j]], rows_v.at[j % 2],
                                    gsem[j % 2])

        def store(j):
            a0 = base + j * SUB
            return pltpu.async_copy(rows_v.at[j % 2],
                                    z_hbm.at[pl.ds(a0, SUB), :], wsem[j % 2])

        g0 = gath(0)
        g1 = gath(1)
        g0.wait()
        w0 = store(0)
        g1.wait()
        w1 = store(1)
        w0.wait()
        g2 = gath(2)
        w1.wait()
        g3 = gath(3)
        g2.wait()
        w2 = store(2)
        g3.wait()
        w3 = store(3)
        w2.wait()
        w3.wait()

    return gather_k


_BC = 512


def _combine_body(z0_ref, z1_ref, sc_ref, out_ref):
    s = sc_ref[...]
    out_ref[...] = s[:, 0:1] * z0_ref[...] + s[:, 1:2] * z1_ref[...]


def _combine(z, scores, interpret=False):
    return pl.pallas_call(
        _combine_body,
        grid=(T // _BC,),
        in_specs=[
            pl.BlockSpec((_BC, H), lambda i: (i, 0)),
            pl.BlockSpec((_BC, H), lambda i: (i + T // _BC, 0)),
            pl.BlockSpec((_BC, TOP_K), lambda i: (i, 0)),
        ],
        out_specs=pl.BlockSpec((_BC, H), lambda i: (i, 0)),
        out_shape=jax.ShapeDtypeStruct((T, H), jnp.float32),
        interpret=interpret,
    )(z, z, scores)


_make_scatter = functools.cache(_make_scatter)
_make_gather = functools.cache(_make_gather)


@jax.jit
def kernel(x, Wg, bg, W1, b1, W2, b2):
    x2 = x.reshape(T, H)
    slot, scores, eob = _gate(x2, Wg, bg)
    slot2 = slot.reshape(A // SUB, SUB)
    xg = _make_scatter()(x2, slot2)
    yg = _ffn(eob, xg, W1, b1.reshape(E, 1, F), W2, b2.reshape(E, 1, H))
    z = _make_gather()(yg, slot2)
    out = _combine(z, scores)
    return out.reshape(B, L, H)
